# Initial kernel scaffold; baseline (speedup 1.0000x reference)
import jax
import jax.numpy as jnp
from jax.experimental import pallas as pl


def kernel(x, edge_index, W1, b1, W2, b2):
    raise NotImplementedError("write your pallas kernel here")


# speedup vs baseline: 15.1424x; 15.1424x over previous
import functools

import jax
import jax.numpy as jnp
from jax import lax
from jax.experimental import pallas as pl
from jax.experimental.pallas import tpu as pltpu
from jax.experimental.pallas import tpu_sc as plsc

NC = 2
NS = 16
NW = NC * NS
CHUNK = 80
STRIPE = 640
NPAD = NS * STRIPE


def _sc_mesh():
    return plsc.VectorSubcoreMesh(core_axis_name="c", subcore_axis_name="s")


@functools.lru_cache(maxsize=None)
def _make_deg(E):
    ept = E // NW
    nchunks = ept // CHUNK

    @functools.partial(
        pl.kernel,
        mesh=_sc_mesh(),
        out_type=jax.ShapeDtypeStruct((NC * NPAD,), jnp.float32),
        scratch_types=[
            pltpu.VMEM((CHUNK,), jnp.int32),
            pltpu.VMEM((CHUNK,), jnp.float32),
            pltpu.VMEM_SHARED((NPAD,), jnp.float32),
        ],
    )
    def deg_kernel(dst_hbm, zeros_hbm, ones_hbm, out_hbm, idx_v, ones_v, acc_sh):
        cid = lax.axis_index("c")
        sid = lax.axis_index("s")
        wid = cid * NS + sid
        pltpu.sync_copy(zeros_hbm, acc_sh.at[pl.ds(sid * STRIPE, STRIPE)])
        pltpu.sync_copy(ones_hbm, ones_v)
        plsc.subcore_barrier()
        base = wid * ept

        def body(i, carry):
            pltpu.sync_copy(dst_hbm.at[pl.ds(base + i * CHUNK, CHUNK)], idx_v)
            pltpu.sync_copy(ones_v, acc_sh.at[idx_v], add=True)
            return carry

        lax.fori_loop(0, nchunks, body, 0)
        plsc.subcore_barrier()
        pltpu.sync_copy(acc_sh.at[pl.ds(sid * STRIPE, STRIPE)],
                        out_hbm.at[pl.ds(cid * NPAD + sid * STRIPE, STRIPE)])

    return deg_kernel


@functools.lru_cache(maxsize=None)
def _make_scatter(E, D):
    ept = E // NW
    nchunks = ept // CHUNK

    @functools.partial(
        pl.kernel,
        mesh=_sc_mesh(),
        out_type=jax.ShapeDtypeStruct((NC * NPAD, D), jnp.float32),
        compiler_params=(None if D % 128 == 0 else
                         pltpu.CompilerParams(use_tc_tiling_on_sc=False)),
        scratch_types=[
            pltpu.VMEM((CHUNK,), jnp.int32),
            pltpu.VMEM((CHUNK,), jnp.int32),
            pltpu.VMEM((CHUNK, D), jnp.float32),
            pltpu.VMEM_SHARED((NPAD, D), jnp.float32),
            pltpu.SemaphoreType.DMA,
        ],
    )
    def scatter_kernel(vals_hbm, src_hbm, dst_hbm, zeros_hbm, out_hbm,
                       sidx_v, didx_v, rows_v, acc_sh, sem):
        cid = lax.axis_index("c")
        sid = lax.axis_index("s")
        wid = cid * NS + sid
        pltpu.sync_copy(zeros_hbm, acc_sh.at[pl.ds(sid * STRIPE, STRIPE)])
        plsc.subcore_barrier()
        base = wid * ept

        def body(i, carry):
            e0 = base + i * CHUNK
            pltpu.sync_copy(src_hbm.at[pl.ds(e0, CHUNK)], sidx_v)
            pltpu.sync_copy(dst_hbm.at[pl.ds(e0, CHUNK)], didx_v)
            pltpu.async_copy(vals_hbm.at[sidx_v], rows_v, sem).wait()
            pltpu.sync_copy(rows_v, acc_sh.at[didx_v], add=True)
            return carry

        lax.fori_loop(0, nchunks, body, 0)
        plsc.subcore_barrier()
        pltpu.sync_copy(acc_sh.at[pl.ds(sid * STRIPE, STRIPE)],
                        out_hbm.at[pl.ds(cid * NPAD + sid * STRIPE, STRIPE)])

    return scatter_kernel


def _tc_prep(d0, d1, x):
    N, Din = x.shape
    R = 1000
    grid = N // R

    def body(d0_ref, d1_ref, x_ref, dinv_ref, x1_ref):
        deg = d0_ref[...] + d1_ref[...] + 1.0
        dinv = lax.rsqrt(deg)
        dinv_ref[...] = dinv
        x1_ref[...] = x_ref[...] * dinv

    return pl.pallas_call(
        body,
        grid=(grid,),
        in_specs=[
            pl.BlockSpec((R, 1), lambda i: (i, 0)),
            pl.BlockSpec((R, 1), lambda i: (i, 0)),
            pl.BlockSpec((R, Din), lambda i: (i, 0)),
        ],
        out_specs=[
            pl.BlockSpec((R, 1), lambda i: (i, 0)),
            pl.BlockSpec((R, Din), lambda i: (i, 0)),
        ],
        out_shape=[
            jax.ShapeDtypeStruct((N, 1), jnp.float32),
            jax.ShapeDtypeStruct((N, Din), jnp.float32),
        ],
    )(d0, d1, x)


def _tc_mid(p0, p1, dinv, x, W1, b1, W2):
    N, Din = x.shape
    Dh = W1.shape[1]
    Do = W2.shape[1]
    R = 1000
    grid = N // R

    def body(p0_ref, p1_ref, dinv_ref, x_ref, w1_ref, b1_ref, w2_ref,
             h2_ref, x2_ref):
        dinv = dinv_ref[...]
        agg = dinv * (p0_ref[...] + p1_ref[...] + dinv * x_ref[...])
        h = jnp.dot(agg, w1_ref[...], preferred_element_type=jnp.float32)
        h = jnp.maximum(h + b1_ref[...], 0.0)
        h2 = jnp.dot(h, w2_ref[...], preferred_element_type=jnp.float32)
        h2_ref[...] = h2
        x2_ref[...] = h2 * dinv

    return pl.pallas_call(
        body,
        grid=(grid,),
        in_specs=[
            pl.BlockSpec((R, Din), lambda i: (i, 0)),
            pl.BlockSpec((R, Din), lambda i: (i, 0)),
            pl.BlockSpec((R, 1), lambda i: (i, 0)),
            pl.BlockSpec((R, Din), lambda i: (i, 0)),
            pl.BlockSpec((Din, Dh), lambda i: (0, 0)),
            pl.BlockSpec((1, Dh), lambda i: (0, 0)),
            pl.BlockSpec((Dh, Do), lambda i: (0, 0)),
        ],
        out_specs=[
            pl.BlockSpec((R, Do), lambda i: (i, 0)),
            pl.BlockSpec((R, Do), lambda i: (i, 0)),
        ],
        out_shape=[
            jax.ShapeDtypeStruct((N, Do), jnp.float32),
            jax.ShapeDtypeStruct((N, Do), jnp.float32),
        ],
    )(p0, p1, dinv, x, W1, b1, W2)


def _tc_final(q0, q1, dinv, h2, b2):
    N, Do = h2.shape
    R = 1000
    grid = N // R

    def body(q0_ref, q1_ref, dinv_ref, h2_ref, b2_ref, out_ref):
        dinv = dinv_ref[...]
        a = dinv * (q0_ref[...] + q1_ref[...] + dinv * h2_ref[...]) + b2_ref[...]
        m = jnp.max(a, axis=1, keepdims=True)
        ex = jnp.exp(a - m)
        lse = jnp.log(jnp.sum(ex, axis=1, keepdims=True))
        out_ref[...] = a - m - lse

    return pl.pallas_call(
        body,
        grid=(grid,),
        in_specs=[
            pl.BlockSpec((R, Do), lambda i: (i, 0)),
            pl.BlockSpec((R, Do), lambda i: (i, 0)),
            pl.BlockSpec((R, 1), lambda i: (i, 0)),
            pl.BlockSpec((R, Do), lambda i: (i, 0)),
            pl.BlockSpec((1, Do), lambda i: (0, 0)),
        ],
        out_specs=pl.BlockSpec((R, Do), lambda i: (i, 0)),
        out_shape=jax.ShapeDtypeStruct((N, Do), jnp.float32),
    )(q0, q1, dinv, h2, b2)


def kernel(x, edge_index, W1, b1, W2, b2):
    N, Din = x.shape
    Do = W2.shape[1]
    E = edge_index.shape[1]
    src = edge_index[0].astype(jnp.int32)
    dst = edge_index[1].astype(jnp.int32)
    epad = (-E) % (NW * CHUNK)
    if epad:
        src = jnp.concatenate([src, jnp.zeros((epad,), jnp.int32)])
        dst = jnp.concatenate([dst, jnp.full((epad,), NPAD - 1, jnp.int32)])
    Ep = E + epad

    degp = _make_deg(Ep)(
        dst,
        jnp.zeros((STRIPE,), jnp.float32),
        jnp.ones((CHUNK,), jnp.float32),
    )
    d0 = degp[:N, None]
    d1 = degp[NPAD:NPAD + N, None]
    dinv, x1 = _tc_prep(d0, d1, x)

    p = _make_scatter(Ep, Din)(
        x1, src, dst, jnp.zeros((STRIPE, Din), jnp.float32))
    h2, x2 = _tc_mid(p[:N], p[NPAD:NPAD + N], dinv, x, W1,
                     b1.reshape(1, -1), W2)

    q = _make_scatter(Ep, Do)(
        x2, src, dst, jnp.zeros((STRIPE, Do), jnp.float32))
    return _tc_final(q[:N], q[NPAD:NPAD + N], dinv, h2, b2.reshape(1, -1))
